# baseline (speedup 1.0000x reference)
import functools
import jax
import jax.numpy as jnp
from jax import lax
from jax.experimental import pallas as pl
from jax.experimental.pallas import tpu as pltpu
from jax.experimental.pallas import tpu_sc as plsc

T = 2048
D = 1024
E = 8
DFF = 512
NG = 4
TG = 2
K = 2
NS = 2
RSF = 2.5

B = 256
NBMAX = 24
ROWS = NBMAX * B
GCH = 64
NWORK = 32

NEG = jnp.finfo(jnp.float32).min
_DN = (((1,), (0,)), ((), ()))
HI = lax.Precision.HIGHEST
DEF = lax.Precision.DEFAULT


def _iota2(shape, dim):
    return lax.broadcasted_iota(jnp.int32, shape, dim)


def _meta_scalar(metav, idx):
    base = (idx // 16) * 16
    chunk = metav[pl.ds(base, 16)]
    lane = lax.iota(jnp.int32, 16)
    return jnp.sum(jnp.where(lane == idx - base, chunk, 0))


def _first_k_mask(vals, k, triu):
    rem = vals
    sel = jnp.zeros_like(vals, dtype=jnp.bool_)
    for _ in range(k):
        m = jnp.max(rem, axis=1, keepdims=True)
        eq = rem == m
        cnt = lax.dot_general(eq.astype(jnp.float32), triu, _DN, precision=HI)
        first = jnp.logical_and(eq, cnt == 1.0)
        sel = jnp.logical_or(sel, first)
        rem = jnp.where(first, NEG, rem)
    return sel


def _routing(x, wg, eb):
    logits = lax.dot_general(x, wg, _DN, precision=DEF)
    scores = jax.nn.sigmoid(logits)
    sfc = scores + eb

    G = (_iota2((E, NG), 0) // (E // NG) == _iota2((E, NG), 1)).astype(
        jnp.float32)
    gsum = lax.dot_general(sfc, G, _DN, precision=HI)
    triu_g = (_iota2((NG, NG), 0) <= _iota2((NG, NG), 1)).astype(jnp.float32)
    gmask = _first_k_mask(gsum, TG, triu_g)
    smask = lax.dot_general(gmask.astype(jnp.float32), G.T, _DN,
                            precision=HI) > 0.5
    masked = jnp.where(smask, sfc, NEG)
    triu_e = (_iota2((E, E), 0) <= _iota2((E, E), 1)).astype(jnp.float32)
    sel = _first_k_mask(masked, K, triu_e)

    w = jnp.where(sel, scores, 0.0)
    wsum = jnp.sum(w, axis=1, keepdims=True) + 1e-20
    return sel, w / wsum * RSF


def _route_body(x_ref, wg_ref, eb_ref, sel_ref, meta_ref, pos_ref,
                wpos_ref, xbf_ref, rank_ref):
    x = x_ref[...]
    xbf_ref[...] = x.astype(jnp.bfloat16)
    sel, comb = _routing(x, wg_ref[...], eb_ref[...])
    sel_f = sel.astype(jnp.float32)
    sel_ref[...] = sel_f

    trilS = (_iota2((B, B), 0) > _iota2((B, B), 1)).astype(jnp.float32)
    running = jnp.zeros((1, E), jnp.float32)
    for c in range(T // B):
        blk = sel_f[c * B:(c + 1) * B, :]
        ex = lax.dot_general(trilS, blk, _DN, precision=HI)
        rank_ref[c * B:(c + 1) * B, :] = ex + running
        running = running + jnp.sum(blk, axis=0, keepdims=True)
    counts = running

    padded = jnp.floor((counts + (B - 1)) * (1.0 / B)) * B
    I8 = (_iota2((E, E), 0) == _iota2((E, E), 1)).astype(jnp.float32)
    L8s = (_iota2((E, E), 0) > _iota2((E, E), 1)).astype(jnp.float32)
    padded_col = lax.dot_general(I8, padded, (((1,), (1,)), ((), ())),
                                 precision=HI)
    bs_col = lax.dot_general(L8s, padded_col, _DN, precision=HI)
    bs_row = lax.dot_general(bs_col, I8, (((0,), (0,)), ((), ())),
                             precision=HI)

    dst = rank_ref[...] + bs_row

    lane_f = _iota2((T, E), 1).astype(jnp.float32)
    e0 = jnp.min(jnp.where(sel, lane_f, 99.0), axis=1, keepdims=True)
    e1 = jnp.max(jnp.where(sel, lane_f, -1.0), axis=1, keepdims=True)
    m0 = lane_f == e0
    m1 = lane_f == e1
    pos0 = jnp.sum(jnp.where(m0, dst, 0.0), axis=1, keepdims=True)
    pos1 = jnp.sum(jnp.where(m1, dst, 0.0), axis=1, keepdims=True)
    w0 = jnp.sum(jnp.where(m0, comb, 0.0), axis=1, keepdims=True)
    w1 = jnp.sum(jnp.where(m1, comb, 0.0), axis=1, keepdims=True)
    pos_ref[...] = jnp.concatenate([pos0, pos1], axis=1).astype(jnp.int32)
    wpos_ref[...] = jnp.concatenate([w0, w1], axis=1)

    bsb_col = bs_col * (1.0 / B)
    Pm = (jnp.broadcast_to(bsb_col, (E, NBMAX)) ==
          _iota2((E, NBMAX), 1).astype(jnp.float32)).astype(jnp.float32)
    ones_row = jnp.ones((1, E), jnp.float32)
    s = lax.dot_general(ones_row, Pm, _DN, precision=HI)
    triu24 = (_iota2((NBMAX, NBMAX), 0) <= _iota2((NBMAX, NBMAX), 1)
              ).astype(jnp.float32)
    ebid = lax.dot_general(s, triu24, _DN, precision=HI) - 1.0
    ebid = jnp.clip(ebid, 0.0, float(E - 1))
    nrows = lax.dot_general(ones_row, padded_col, _DN, precision=HI)
    nblocks = nrows * (1.0 / B)
    valid = (_iota2((1, NBMAX), 1).astype(jnp.float32) <
             jnp.broadcast_to(nblocks, (1, NBMAX))).astype(jnp.float32)
    meta_f = jnp.concatenate([
        bs_row,
        padded,
        nrows,
        nblocks,
        jnp.zeros((1, 6), jnp.float32),
        ebid,
        valid,
        jnp.zeros((1, 128 - 72), jnp.float32),
    ], axis=1)
    meta_ref[...] = meta_f.astype(jnp.int32)


@jax.jit
def _route(x, wg, eb2):
    return pl.pallas_call(
        _route_body,
        grid=(1,),
        in_specs=[
            pl.BlockSpec((T, D), lambda i: (0, 0)),
            pl.BlockSpec((D, E), lambda i: (0, 0)),
            pl.BlockSpec((1, E), lambda i: (0, 0)),
        ],
        out_specs=[
            pl.BlockSpec((T, E), lambda i: (0, 0)),
            pl.BlockSpec((1, 128), lambda i: (0, 0)),
            pl.BlockSpec((T, 2), lambda i: (0, 0)),
            pl.BlockSpec((T, 2), lambda i: (0, 0)),
            pl.BlockSpec((T, D), lambda i: (0, 0)),
        ],
        out_shape=[
            jax.ShapeDtypeStruct((T, E), jnp.float32),
            jax.ShapeDtypeStruct((1, 128), jnp.int32),
            jax.ShapeDtypeStruct((T, 2), jnp.int32),
            jax.ShapeDtypeStruct((T, 2), jnp.float32),
            jax.ShapeDtypeStruct((T, D), jnp.bfloat16),
        ],
        scratch_shapes=[pltpu.VMEM((T, E), jnp.float32)],
    )(x, wg, eb2)


@functools.cache
def _vmesh():
    return plsc.VectorSubcoreMesh(core_axis_name="c", subcore_axis_name="s")


@jax.jit
def _sc_compact(selT, meta):
    @functools.partial(
        pl.kernel,
        out_type=jax.ShapeDtypeStruct((ROWS,), jnp.int32),
        mesh=_vmesh(),
        scratch_types=[
            pltpu.VMEM((T,), jnp.float32),
            pltpu.VMEM((T + 16,), jnp.int32),
            pltpu.VMEM((128,), jnp.int32),
            pltpu.SemaphoreType.DMA,
        ],
        compiler_params=pltpu.CompilerParams(needs_layout_passes=False),
    )
    def kern(selT_hbm, meta_hbm, tok_hbm, cvals, toks, meta_s, sem):
        wid = lax.axis_index("s") * 2 + lax.axis_index("c")
        pltpu.sync_copy(meta_hbm.at[0], meta_s)

        @pl.when(wid < E)
        def _():
            e = wid
            pltpu.sync_copy(selT_hbm.at[e], cvals)

            @pl.loop(0, T + 16, step=16)
            def _(i):
                toks[pl.ds(i, 16)] = jnp.zeros((16,), jnp.int32)

            def step(c, cnt):
                v = cvals[pl.ds(c * 16, 16)]
                m = v != 0.0
                tk = lax.iota(jnp.int32, 16) + c * 16
                plsc.store_compressed(toks.at[pl.ds(cnt, 16)], tk, mask=m)
                return cnt + jnp.sum(m.astype(jnp.int32))

            lax.fori_loop(0, T // 16, step, 0)

            bs = _meta_scalar(meta_s, e)
            nch = _meta_scalar(meta_s, 8 + e) // B

            def out_step(j, _):
                off = pl.multiple_of(bs + j * B, B)
                pltpu.sync_copy(toks.at[pl.ds(j * B, B)],
                                tok_hbm.at[pl.ds(off, B)])
                return 0

            lax.fori_loop(0, nch, out_step, 0)

    return kern(selT, meta)


def _make_sc_gather(n_out, max_chunks, meta_chunks):
    iters = (max_chunks + NWORK - 1) // NWORK
    CW = D // 2

    @jax.jit
    def run(src, idx, meta):
        @functools.partial(
            pl.kernel,
            out_type=jax.ShapeDtypeStruct((n_out, CW), jnp.int32),
            mesh=_vmesh(),
            scratch_types=[
                pltpu.VMEM((iters, GCH), jnp.int32),
                pltpu.VMEM((iters, GCH, CW), jnp.int32),
                pltpu.VMEM((128,), jnp.int32),
            ] + [pltpu.SemaphoreType.DMA] * iters,
            compiler_params=pltpu.CompilerParams(needs_layout_passes=False),
        )
        def kern(src_hbm, idx_hbm, meta_hbm, out_hbm, idxv, bufs, meta_s,
                 *sems):
            wid = lax.axis_index("s") * 2 + lax.axis_index("c")
            pltpu.sync_copy(meta_hbm.at[0], meta_s)
            if meta_chunks:
                nch = _meta_scalar(meta_s, 17) * (B // GCH)
            else:
                nch = max_chunks
            for it in range(iters):
                c = wid + it * NWORK

                @pl.when(c < nch)
                def _(it=it, c=c):
                    pltpu.sync_copy(idx_hbm.at[pl.ds(c * GCH, GCH)],
                                    idxv.at[it])
                    pltpu.async_copy(src_hbm.at[idxv.at[it]], bufs.at[it],
                                     sems[it])
            for it in range(iters):
                c = wid + it * NWORK

                @pl.when(c < nch)
                def _(it=it, c=c):
                    pltpu.make_async_copy(src_hbm.at[idxv.at[it]],
                                          bufs.at[it], sems[it]).wait()
                    pltpu.sync_copy(bufs.at[it],
                                    out_hbm.at[pl.ds(c * GCH, GCH)])

        return kern(src, idx, meta)

    return run


_sc_gather_x = _make_sc_gather(ROWS, ROWS // GCH, True)
_sc_gather_y = _make_sc_gather(K * T, (K * T) // GCH, False)


def _mm_body(ebid_ref, valid_ref, xg_ref, wgu_ref, wd_ref, yg_ref):
    b = pl.program_id(0)

    @pl.when(valid_ref[b] == 1)
    def _():
        xb = xg_ref[...].astype(jnp.float32)
        gu = lax.dot_general(xb, wgu_ref[0], _DN, precision=DEF)
        g = gu[:, :DFF]
        u = gu[:, DFF:]
        h = g * jax.nn.sigmoid(g) * u
        yg_ref[...] = lax.dot_general(
            h, wd_ref[0], _DN, precision=DEF).astype(jnp.bfloat16)


@jax.jit
def _mm(ebid, valid, xg, W_gate_up, W_down):
    grid_spec = pltpu.PrefetchScalarGridSpec(
        num_scalar_prefetch=2,
        grid=(NBMAX,),
        in_specs=[
            pl.BlockSpec((B, D), lambda b, er, vr: (b, 0)),
            pl.BlockSpec((1, D, 2 * DFF), lambda b, er, vr: (er[b], 0, 0)),
            pl.BlockSpec((1, DFF, D), lambda b, er, vr: (er[b], 0, 0)),
        ],
        out_specs=pl.BlockSpec((B, D), lambda b, er, vr: (b, 0)),
    )
    return pl.pallas_call(
        _mm_body,
        grid_spec=grid_spec,
        out_shape=jax.ShapeDtypeStruct((ROWS, D), jnp.bfloat16),
        compiler_params=pltpu.CompilerParams(
            dimension_semantics=("arbitrary",)),
    )(ebid, valid, xg, W_gate_up, W_down)


TBS = 1024


def _sh_body(x_ref, wsg_ref, wsu_ref, wds_ref, out_ref):
    pe = pl.program_id(1)
    g = lax.dot_general(x_ref[...], wsg_ref[...], _DN, precision=DEF)
    u = lax.dot_general(x_ref[...], wsu_ref[...], _DN, precision=DEF)
    h = g * jax.nn.sigmoid(g) * u
    o = lax.dot_general(h, wds_ref[0], _DN, precision=DEF)

    @pl.when(pe == 0)
    def _():
        out_ref[...] = o

    @pl.when(pe != 0)
    def _():
        out_ref[...] += o


@jax.jit
def _shared(x, Ws_gate_up, Ws_down3):
    return pl.pallas_call(
        _sh_body,
        grid=(T // TBS, NS),
        in_specs=[
            pl.BlockSpec((TBS, D), lambda t, pe: (t, 0)),
            pl.BlockSpec((D, DFF), lambda t, pe: (0, pe)),
            pl.BlockSpec((D, DFF), lambda t, pe: (0, pe + NS)),
            pl.BlockSpec((1, DFF, D), lambda t, pe: (pe, 0, 0)),
        ],
        out_specs=pl.BlockSpec((TBS, D), lambda t, pe: (t, 0)),
        out_shape=jax.ShapeDtypeStruct((T, D), jnp.float32),
        compiler_params=pltpu.CompilerParams(
            dimension_semantics=("arbitrary", "arbitrary")),
    )(x, Ws_gate_up, Ws_gate_up, Ws_down3)


def _fin_body(y0_ref, y1_ref, sh_ref, w0_ref, w1_ref, out_ref):
    y0 = y0_ref[...].astype(jnp.float32)
    y1 = y1_ref[...].astype(jnp.float32)
    out_ref[...] = w0_ref[...] * y0 + w1_ref[...] * y1 + sh_ref[...]


@jax.jit
def _final(y01, sh, w0, w1):
    return pl.pallas_call(
        _fin_body,
        grid=(T // TBS,),
        in_specs=[
            pl.BlockSpec((TBS, D), lambda t: (t, 0)),
            pl.BlockSpec((TBS, D), lambda t: (t + T // TBS, 0)),
            pl.BlockSpec((TBS, D), lambda t: (t, 0)),
            pl.BlockSpec((TBS, 1), lambda t: (t, 0)),
            pl.BlockSpec((TBS, 1), lambda t: (t, 0)),
        ],
        out_specs=pl.BlockSpec((TBS, D), lambda t: (t, 0)),
        out_shape=jax.ShapeDtypeStruct((T, D), jnp.float32),
    )(y01, y01, sh, w0, w1)


def kernel(hidden_states, W_gate, e_bias, W_gate_up, W_down, Ws_gate_up,
           Ws_down):
    x = hidden_states
    eb2 = e_bias.reshape(1, E)
    Ws_down3 = Ws_down.reshape(NS, DFF, D)

    sel, meta, pos, wpos, xbf = _route(x, W_gate, eb2)
    selT = sel.T
    ebid = meta[0, 24:48]
    valid = meta[0, 48:72]
    posflat = pos.T.reshape(K * T)
    w0 = wpos[:, 0:1]
    w1 = wpos[:, 1:2]

    tok_map = _sc_compact(selT, meta)
    xbf32 = lax.bitcast_convert_type(
        xbf.reshape(T, D // 2, 2), jnp.int32)
    xg32 = _sc_gather_x(xbf32, tok_map, meta)
    xg = lax.bitcast_convert_type(xg32, jnp.bfloat16).reshape(ROWS, D)
    yg = _mm(ebid, valid, xg, W_gate_up, W_down)
    yg32 = lax.bitcast_convert_type(
        yg.reshape(ROWS, D // 2, 2), jnp.int32)
    y32 = _sc_gather_y(yg32, posflat, meta)
    y01 = lax.bitcast_convert_type(y32, jnp.bfloat16).reshape(K * T, D)
    sh = _shared(x, Ws_gate_up, Ws_down3)
    return _final(y01, sh, w0, w1)


# speedup vs baseline: 5.7394x; 5.7394x over previous
import functools
import jax
import jax.numpy as jnp
from jax import lax
from jax.experimental import pallas as pl
from jax.experimental.pallas import tpu as pltpu

T = 2048
D = 1024
E = 8
DFF = 512
NG = 4
TG = 2
K = 2
NS = 2
RSF = 2.5

NEG = jnp.finfo(jnp.float32).min


def _first_k_mask(vals, k, triu):
    n = vals.shape[1]
    rem = vals
    sel = jnp.zeros_like(vals, dtype=jnp.bool_)
    for _ in range(k):
        m = jnp.max(rem, axis=1, keepdims=True)
        eq = rem == m
        cnt = lax.dot_general(
            eq.astype(jnp.float32), triu,
            (((1,), (0,)), ((), ())),
            precision=lax.Precision.HIGHEST,
        )
        first = jnp.logical_and(eq, cnt == 1.0)
        sel = jnp.logical_or(sel, first)
        rem = jnp.where(first, NEG, rem)
    return sel


def _routing(x, wg, eb):
    logits = lax.dot_general(
        x, wg, (((1,), (0,)), ((), ())), precision=lax.Precision.DEFAULT)
    scores = jax.nn.sigmoid(logits)
    sfc = scores + eb

    r8 = lax.broadcasted_iota(jnp.int32, (E, NG), 0)
    c8 = lax.broadcasted_iota(jnp.int32, (E, NG), 1)
    G = (r8 // (E // NG) == c8).astype(jnp.float32)
    gsum = lax.dot_general(
        sfc, G, (((1,), (0,)), ((), ())), precision=lax.Precision.HIGHEST)

    rg = lax.broadcasted_iota(jnp.int32, (NG, NG), 0)
    cg = lax.broadcasted_iota(jnp.int32, (NG, NG), 1)
    triu_g = (rg <= cg).astype(jnp.float32)
    gmask = _first_k_mask(gsum, TG, triu_g)

    smask = lax.dot_general(
        gmask.astype(jnp.float32), G.T, (((1,), (0,)), ((), ())),
        precision=lax.Precision.HIGHEST) > 0.5
    masked = jnp.where(smask, sfc, NEG)

    re_ = lax.broadcasted_iota(jnp.int32, (E, E), 0)
    ce_ = lax.broadcasted_iota(jnp.int32, (E, E), 1)
    triu_e = (re_ <= ce_).astype(jnp.float32)
    sel = _first_k_mask(masked, K, triu_e)

    w = jnp.where(sel, scores, 0.0)
    wsum = jnp.sum(w, axis=1, keepdims=True) + 1e-20
    return w / wsum * RSF


TB = 2048


def _moe_body(x_ref, wg_ref, eb_ref, wgur_ref, wsg_ref, wsu_ref,
              wdr_ref, wds_ref, out_ref, comb_ref):
    e = pl.program_id(1)

    @pl.when(e == 0)
    def _():
        comb_ref[...] = _routing(x_ref[...], wg_ref[...], eb_ref[...])
        out_ref[...] = jnp.zeros_like(out_ref)

    x = x_ref[...]

    @pl.when(e < E)
    def _():
        gu = lax.dot_general(
            x, wgur_ref[0], (((1,), (0,)), ((), ())),
            precision=lax.Precision.DEFAULT)
        g = gu[:, :DFF]
        u = gu[:, DFF:]
        lane = lax.broadcasted_iota(jnp.int32, (TB, E), 1)
        wsel = jnp.sum(jnp.where(lane == e, comb_ref[...], 0.0),
                       axis=1, keepdims=True)
        h = g * jax.nn.sigmoid(g) * u * wsel
        out_ref[...] += lax.dot_general(
            h, wdr_ref[0], (((1,), (0,)), ((), ())),
            precision=lax.Precision.DEFAULT)

    @pl.when(e >= E)
    def _():
        g = lax.dot_general(
            x, wsg_ref[...], (((1,), (0,)), ((), ())),
            precision=lax.Precision.DEFAULT)
        u = lax.dot_general(
            x, wsu_ref[...], (((1,), (0,)), ((), ())),
            precision=lax.Precision.DEFAULT)
        h = g * jax.nn.sigmoid(g) * u
        out_ref[...] += lax.dot_general(
            h, wds_ref[0], (((1,), (0,)), ((), ())),
            precision=lax.Precision.DEFAULT)


@jax.jit
def _moe(hidden_states, W_gate, e_bias2, W_gate_up, W_down, Ws_gate_up,
         Ws_down3):
    grid = (T // TB, E + NS)
    clamp_r = lambda t, e: jnp.minimum(e, E - 1)
    clamp_s = lambda t, e: jnp.clip(e - E, 0, NS - 1)
    return pl.pallas_call(
        _moe_body,
        grid=grid,
        in_specs=[
            pl.BlockSpec((TB, D), lambda t, e: (t, 0)),
            pl.BlockSpec((D, E), lambda t, e: (0, 0)),
            pl.BlockSpec((1, E), lambda t, e: (0, 0)),
            pl.BlockSpec((1, D, 2 * DFF),
                         lambda t, e: (clamp_r(t, e), 0, 0)),
            pl.BlockSpec((D, DFF),
                         lambda t, e: (0, clamp_s(t, e))),
            pl.BlockSpec((D, DFF),
                         lambda t, e: (0, clamp_s(t, e) + NS)),
            pl.BlockSpec((1, DFF, D),
                         lambda t, e: (clamp_r(t, e), 0, 0)),
            pl.BlockSpec((1, DFF, D),
                         lambda t, e: (clamp_s(t, e), 0, 0)),
        ],
        out_specs=pl.BlockSpec((TB, D), lambda t, e: (t, 0)),
        out_shape=jax.ShapeDtypeStruct((T, D), jnp.float32),
        scratch_shapes=[pltpu.VMEM((TB, E), jnp.float32)],
        compiler_params=pltpu.CompilerParams(
            dimension_semantics=("arbitrary", "arbitrary"),
            vmem_limit_bytes=64 * 1024 * 1024),
    )(hidden_states, W_gate, e_bias2, W_gate_up, Ws_gate_up, Ws_gate_up,
      W_down, Ws_down3)


def kernel(hidden_states, W_gate, e_bias, W_gate_up, W_down, Ws_gate_up,
           Ws_down):
    e_bias2 = e_bias.reshape(1, E)
    Ws_down3 = Ws_down.reshape(NS, DFF, D)
    return _moe(hidden_states, W_gate, e_bias2, W_gate_up, W_down,
                Ws_gate_up, Ws_down3)


# speedup vs baseline: 5.8642x; 1.0217x over previous
import functools
import jax
import jax.numpy as jnp
from jax import lax
from jax.experimental import pallas as pl
from jax.experimental.pallas import tpu as pltpu

T = 2048
D = 1024
E = 8
DFF = 512
NG = 4
TG = 2
K = 2
NS = 2
RSF = 2.5

NEG = jnp.finfo(jnp.float32).min


def _first_k_mask(vals, k, triu):
    n = vals.shape[1]
    rem = vals
    sel = jnp.zeros_like(vals, dtype=jnp.bool_)
    for _ in range(k):
        m = jnp.max(rem, axis=1, keepdims=True)
        eq = rem == m
        cnt = lax.dot_general(
            eq.astype(jnp.float32), triu,
            (((1,), (0,)), ((), ())),
            precision=lax.Precision.HIGHEST,
        )
        first = jnp.logical_and(eq, cnt == 1.0)
        sel = jnp.logical_or(sel, first)
        rem = jnp.where(first, NEG, rem)
    return sel


def _routing(x, wg, eb):
    logits = lax.dot_general(
        x, wg, (((1,), (0,)), ((), ())), precision=lax.Precision.DEFAULT)
    scores = jax.nn.sigmoid(logits)
    sfc = scores + eb

    r8 = lax.broadcasted_iota(jnp.int32, (E, NG), 0)
    c8 = lax.broadcasted_iota(jnp.int32, (E, NG), 1)
    G = (r8 // (E // NG) == c8).astype(jnp.float32)
    gsum = lax.dot_general(
        sfc, G, (((1,), (0,)), ((), ())), precision=lax.Precision.HIGHEST)

    rg = lax.broadcasted_iota(jnp.int32, (NG, NG), 0)
    cg = lax.broadcasted_iota(jnp.int32, (NG, NG), 1)
    triu_g = (rg <= cg).astype(jnp.float32)
    gmask = _first_k_mask(gsum, TG, triu_g)

    smask = lax.dot_general(
        gmask.astype(jnp.float32), G.T, (((1,), (0,)), ((), ())),
        precision=lax.Precision.HIGHEST) > 0.5
    masked = jnp.where(smask, sfc, NEG)

    re_ = lax.broadcasted_iota(jnp.int32, (E, E), 0)
    ce_ = lax.broadcasted_iota(jnp.int32, (E, E), 1)
    triu_e = (re_ <= ce_).astype(jnp.float32)
    sel = _first_k_mask(masked, K, triu_e)

    w = jnp.where(sel, scores, 0.0)
    wsum = jnp.sum(w, axis=1, keepdims=True) + 1e-20
    return w / wsum * RSF


TB = 1024


def _bdot(a, b):
    return lax.dot_general(a, b, (((1,), (0,)), ((), ())),
                           preferred_element_type=jnp.float32)


def _moe_body(x_ref, wg_ref, eb_ref, wgur_ref, wsg_ref, wsu_ref,
              wdr_ref, wds_ref, out_ref, comb_ref, xb_ref):
    e = pl.program_id(1)

    @pl.when(e == 0)
    def _():
        comb_ref[...] = _routing(x_ref[...], wg_ref[...], eb_ref[...])
        xb_ref[...] = x_ref[...].astype(jnp.bfloat16)
        out_ref[...] = jnp.zeros_like(out_ref)

    xb = xb_ref[...]

    @pl.when(e < E)
    def _():
        gu = _bdot(xb, wgur_ref[0].astype(jnp.bfloat16))
        g = gu[:, :DFF]
        u = gu[:, DFF:]
        lane = lax.broadcasted_iota(jnp.int32, (TB, E), 1)
        wsel = jnp.sum(jnp.where(lane == e, comb_ref[...], 0.0),
                       axis=1, keepdims=True)
        h = (g * jax.nn.sigmoid(g) * u * wsel).astype(jnp.bfloat16)
        out_ref[...] += _bdot(h, wdr_ref[0].astype(jnp.bfloat16))

    @pl.when(e >= E)
    def _():
        g = _bdot(xb, wsg_ref[...].astype(jnp.bfloat16))
        u = _bdot(xb, wsu_ref[...].astype(jnp.bfloat16))
        h = (g * jax.nn.sigmoid(g) * u).astype(jnp.bfloat16)
        out_ref[...] += _bdot(h, wds_ref[0].astype(jnp.bfloat16))


@jax.jit
def _moe(hidden_states, W_gate, e_bias2, W_gate_up, W_down, Ws_gate_up,
         Ws_down3):
    grid = (T // TB, E + NS)
    clamp_r = lambda t, e: jnp.minimum(e, E - 1)
    clamp_s = lambda t, e: jnp.clip(e - E, 0, NS - 1)
    return pl.pallas_call(
        _moe_body,
        grid=grid,
        in_specs=[
            pl.BlockSpec((TB, D), lambda t, e: (t, 0)),
            pl.BlockSpec((D, E), lambda t, e: (0, 0)),
            pl.BlockSpec((1, E), lambda t, e: (0, 0)),
            pl.BlockSpec((1, D, 2 * DFF),
                         lambda t, e: (clamp_r(t, e), 0, 0)),
            pl.BlockSpec((D, DFF),
                         lambda t, e: (0, clamp_s(t, e))),
            pl.BlockSpec((D, DFF),
                         lambda t, e: (0, clamp_s(t, e) + NS)),
            pl.BlockSpec((1, DFF, D),
                         lambda t, e: (clamp_r(t, e), 0, 0)),
            pl.BlockSpec((1, DFF, D),
                         lambda t, e: (clamp_s(t, e), 0, 0)),
        ],
        out_specs=pl.BlockSpec((TB, D), lambda t, e: (t, 0)),
        out_shape=jax.ShapeDtypeStruct((T, D), jnp.float32),
        scratch_shapes=[pltpu.VMEM((TB, E), jnp.float32),
                        pltpu.VMEM((TB, D), jnp.bfloat16)],
        compiler_params=pltpu.CompilerParams(
            dimension_semantics=("arbitrary", "arbitrary"),
            vmem_limit_bytes=64 * 1024 * 1024),
    )(hidden_states, W_gate, e_bias2, W_gate_up, Ws_gate_up, Ws_gate_up,
      W_down, Ws_down3)


def kernel(hidden_states, W_gate, e_bias, W_gate_up, W_down, Ws_gate_up,
           Ws_down):
    e_bias2 = e_bias.reshape(1, E)
    Ws_down3 = Ws_down.reshape(NS, DFF, D)
    return _moe(hidden_states, W_gate, e_bias2, W_gate_up, W_down,
                Ws_gate_up, Ws_down3)


# speedup vs baseline: 6.1384x; 1.0468x over previous
import functools
import jax
import jax.numpy as jnp
from jax import lax
from jax.experimental import pallas as pl
from jax.experimental.pallas import tpu as pltpu

T = 2048
D = 1024
E = 8
DFF = 512
NG = 4
TG = 2
K = 2
NS = 2
RSF = 2.5

NEG = jnp.finfo(jnp.float32).min


def _first_k_mask(vals, k, triu):
    n = vals.shape[1]
    rem = vals
    sel = jnp.zeros_like(vals, dtype=jnp.bool_)
    for _ in range(k):
        m = jnp.max(rem, axis=1, keepdims=True)
        eq = rem == m
        cnt = lax.dot_general(
            eq.astype(jnp.float32), triu,
            (((1,), (0,)), ((), ())),
            precision=lax.Precision.HIGHEST,
        )
        first = jnp.logical_and(eq, cnt == 1.0)
        sel = jnp.logical_or(sel, first)
        rem = jnp.where(first, NEG, rem)
    return sel


def _routing(x, wg, eb):
    logits = lax.dot_general(
        x, wg, (((1,), (0,)), ((), ())), precision=lax.Precision.DEFAULT)
    scores = jax.nn.sigmoid(logits)
    sfc = scores + eb

    r8 = lax.broadcasted_iota(jnp.int32, (E, NG), 0)
    c8 = lax.broadcasted_iota(jnp.int32, (E, NG), 1)
    G = (r8 // (E // NG) == c8).astype(jnp.float32)
    gsum = lax.dot_general(
        sfc, G, (((1,), (0,)), ((), ())), precision=lax.Precision.HIGHEST)

    rg = lax.broadcasted_iota(jnp.int32, (NG, NG), 0)
    cg = lax.broadcasted_iota(jnp.int32, (NG, NG), 1)
    triu_g = (rg <= cg).astype(jnp.float32)
    gmask = _first_k_mask(gsum, TG, triu_g)

    smask = lax.dot_general(
        gmask.astype(jnp.float32), G.T, (((1,), (0,)), ((), ())),
        precision=lax.Precision.HIGHEST) > 0.5
    masked = jnp.where(smask, sfc, NEG)

    re_ = lax.broadcasted_iota(jnp.int32, (E, E), 0)
    ce_ = lax.broadcasted_iota(jnp.int32, (E, E), 1)
    triu_e = (re_ <= ce_).astype(jnp.float32)
    sel = _first_k_mask(masked, K, triu_e)

    w = jnp.where(sel, scores, 0.0)
    wsum = jnp.sum(w, axis=1, keepdims=True) + 1e-20
    return w / wsum * RSF


TB = 1024


def _bdot(a, b):
    return lax.dot_general(a, b, (((1,), (0,)), ((), ())),
                           preferred_element_type=jnp.float32)


def _moe_body(x_ref, wg_ref, eb_ref, wgur_ref, wsg_ref, wsu_ref,
              wdr_ref, wds_ref, out_ref, comb_ref, xb_ref):
    p = pl.program_id(1)

    @pl.when(p == 0)
    def _():
        comb_ref[...] = _routing(x_ref[...], wg_ref[...], eb_ref[...])
        xb_ref[...] = x_ref[...].astype(jnp.bfloat16)
        out_ref[...] = jnp.zeros_like(out_ref)

    xb = xb_ref[...]
    lane = lax.broadcasted_iota(jnp.int32, (TB, E), 1)

    @pl.when(p < E // 2)
    def _():
        gu0 = _bdot(xb, wgur_ref[0].astype(jnp.bfloat16))
        gu1 = _bdot(xb, wgur_ref[1].astype(jnp.bfloat16))
        w0 = jnp.sum(jnp.where(lane == 2 * p, comb_ref[...], 0.0),
                     axis=1, keepdims=True)
        w1 = jnp.sum(jnp.where(lane == 2 * p + 1, comb_ref[...], 0.0),
                     axis=1, keepdims=True)
        g0 = gu0[:, :DFF]
        u0 = gu0[:, DFF:]
        g1 = gu1[:, :DFF]
        u1 = gu1[:, DFF:]
        h0 = (g0 * jax.nn.sigmoid(g0) * u0 * w0).astype(jnp.bfloat16)
        h1 = (g1 * jax.nn.sigmoid(g1) * u1 * w1).astype(jnp.bfloat16)
        acc = _bdot(h0, wdr_ref[0].astype(jnp.bfloat16))
        acc = acc + _bdot(h1, wdr_ref[1].astype(jnp.bfloat16))
        out_ref[...] += acc

    @pl.when(p >= E // 2)
    def _():
        g0 = _bdot(xb, wsg_ref[:, :DFF].astype(jnp.bfloat16))
        u0 = _bdot(xb, wsu_ref[:, :DFF].astype(jnp.bfloat16))
        g1 = _bdot(xb, wsg_ref[:, DFF:].astype(jnp.bfloat16))
        u1 = _bdot(xb, wsu_ref[:, DFF:].astype(jnp.bfloat16))
        h0 = (g0 * jax.nn.sigmoid(g0) * u0).astype(jnp.bfloat16)
        h1 = (g1 * jax.nn.sigmoid(g1) * u1).astype(jnp.bfloat16)
        acc = _bdot(h0, wds_ref[0].astype(jnp.bfloat16))
        acc = acc + _bdot(h1, wds_ref[1].astype(jnp.bfloat16))
        out_ref[...] += acc


@jax.jit
def _moe(hidden_states, W_gate, e_bias2, W_gate_up, W_down, Ws_gate_up,
         Ws_down3):
    grid = (T // TB, E // 2 + 1)
    cr = lambda t, p: jnp.minimum(p, E // 2 - 1)
    return pl.pallas_call(
        _moe_body,
        grid=grid,
        in_specs=[
            pl.BlockSpec((TB, D), lambda t, p: (t, 0)),
            pl.BlockSpec((D, E), lambda t, p: (0, 0)),
            pl.BlockSpec((1, E), lambda t, p: (0, 0)),
            pl.BlockSpec((2, D, 2 * DFF),
                         lambda t, p: (cr(t, p), 0, 0)),
            pl.BlockSpec((D, 2 * DFF), lambda t, p: (0, 0)),
            pl.BlockSpec((D, 2 * DFF), lambda t, p: (0, 1)),
            pl.BlockSpec((2, DFF, D),
                         lambda t, p: (cr(t, p), 0, 0)),
            pl.BlockSpec((2, DFF, D), lambda t, p: (0, 0, 0)),
        ],
        out_specs=pl.BlockSpec((TB, D), lambda t, p: (t, 0)),
        out_shape=jax.ShapeDtypeStruct((T, D), jnp.float32),
        scratch_shapes=[pltpu.VMEM((TB, E), jnp.float32),
                        pltpu.VMEM((TB, D), jnp.bfloat16)],
        compiler_params=pltpu.CompilerParams(
            dimension_semantics=("arbitrary", "arbitrary"),
            vmem_limit_bytes=64 * 1024 * 1024),
    )(hidden_states, W_gate, e_bias2, W_gate_up,
      Ws_gate_up, Ws_gate_up, W_down, Ws_down3)


def kernel(hidden_states, W_gate, e_bias, W_gate_up, W_down, Ws_gate_up,
           Ws_down):
    e_bias2 = e_bias.reshape(1, E)
    Ws_down3 = Ws_down.reshape(NS, DFF, D)
    return _moe(hidden_states, W_gate, e_bias2, W_gate_up, W_down,
                Ws_gate_up, Ws_down3)


# speedup vs baseline: 6.2285x; 1.0147x over previous
import functools
import jax
import jax.numpy as jnp
from jax import lax
from jax.experimental import pallas as pl
from jax.experimental.pallas import tpu as pltpu

T = 2048
D = 1024
E = 8
DFF = 512
NG = 4
TG = 2
K = 2
NS = 2
RSF = 2.5

NEG = jnp.finfo(jnp.float32).min


def _first_k_mask(vals, k, triu):
    n = vals.shape[1]
    rem = vals
    sel = jnp.zeros_like(vals, dtype=jnp.bool_)
    for _ in range(k):
        m = jnp.max(rem, axis=1, keepdims=True)
        eq = rem == m
        cnt = lax.dot_general(
            eq.astype(jnp.float32), triu,
            (((1,), (0,)), ((), ())),
            precision=lax.Precision.HIGHEST,
        )
        first = jnp.logical_and(eq, cnt == 1.0)
        sel = jnp.logical_or(sel, first)
        rem = jnp.where(first, NEG, rem)
    return sel


def _routing(x, wg, eb):
    logits = lax.dot_general(
        x, wg, (((1,), (0,)), ((), ())), precision=lax.Precision.DEFAULT)
    scores = jax.nn.sigmoid(logits)
    sfc = scores + eb

    r8 = lax.broadcasted_iota(jnp.int32, (E, NG), 0)
    c8 = lax.broadcasted_iota(jnp.int32, (E, NG), 1)
    G = (r8 // (E // NG) == c8).astype(jnp.float32)
    gsum = lax.dot_general(
        sfc, G, (((1,), (0,)), ((), ())), precision=lax.Precision.HIGHEST)

    rg = lax.broadcasted_iota(jnp.int32, (NG, NG), 0)
    cg = lax.broadcasted_iota(jnp.int32, (NG, NG), 1)
    triu_g = (rg <= cg).astype(jnp.float32)
    gmask = _first_k_mask(gsum, TG, triu_g)

    smask = lax.dot_general(
        gmask.astype(jnp.float32), G.T, (((1,), (0,)), ((), ())),
        precision=lax.Precision.HIGHEST) > 0.5
    masked = jnp.where(smask, sfc, NEG)

    re_ = lax.broadcasted_iota(jnp.int32, (E, E), 0)
    ce_ = lax.broadcasted_iota(jnp.int32, (E, E), 1)
    triu_e = (re_ <= ce_).astype(jnp.float32)
    sel = _first_k_mask(masked, K, triu_e)

    w = jnp.where(sel, scores, 0.0)
    wsum = jnp.sum(w, axis=1, keepdims=True) + 1e-20
    return w / wsum * RSF


TB = 1024


def _bdot(a, b):
    return lax.dot_general(a, b, (((1,), (0,)), ((), ())),
                           preferred_element_type=jnp.float32)


def _moe_body(x_ref, wg_ref, eb_ref, wgur_ref, wsg_ref, wsu_ref,
              wdr_ref, wds_ref, out_ref, comb_ref, xb_ref):
    p = pl.program_id(1)

    @pl.when(p == 0)
    def _():
        xb0 = x_ref[...].astype(jnp.bfloat16)
        xb_ref[...] = xb0
        g0 = _bdot(xb0, wsg_ref[:, :DFF].astype(jnp.bfloat16))
        u0 = _bdot(xb0, wsu_ref[:, :DFF].astype(jnp.bfloat16))
        g1 = _bdot(xb0, wsg_ref[:, DFF:].astype(jnp.bfloat16))
        u1 = _bdot(xb0, wsu_ref[:, DFF:].astype(jnp.bfloat16))
        comb_ref[...] = _routing(x_ref[...], wg_ref[...], eb_ref[...])
        h0 = (g0 * jax.nn.sigmoid(g0) * u0).astype(jnp.bfloat16)
        h1 = (g1 * jax.nn.sigmoid(g1) * u1).astype(jnp.bfloat16)
        acc = _bdot(h0, wds_ref[0].astype(jnp.bfloat16))
        acc = acc + _bdot(h1, wds_ref[1].astype(jnp.bfloat16))
        out_ref[...] = acc

    xb = xb_ref[...]
    lane = lax.broadcasted_iota(jnp.int32, (TB, E), 1)

    @pl.when(p >= 1)
    def _():
        gu0 = _bdot(xb, wgur_ref[0].astype(jnp.bfloat16))
        gu1 = _bdot(xb, wgur_ref[1].astype(jnp.bfloat16))
        w0 = jnp.sum(jnp.where(lane == 2 * (p - 1), comb_ref[...], 0.0),
                     axis=1, keepdims=True)
        w1 = jnp.sum(jnp.where(lane == 2 * (p - 1) + 1, comb_ref[...], 0.0),
                     axis=1, keepdims=True)
        g0 = gu0[:, :DFF]
        u0 = gu0[:, DFF:]
        g1 = gu1[:, :DFF]
        u1 = gu1[:, DFF:]
        h0 = (g0 * jax.nn.sigmoid(g0) * u0 * w0).astype(jnp.bfloat16)
        h1 = (g1 * jax.nn.sigmoid(g1) * u1 * w1).astype(jnp.bfloat16)
        acc = _bdot(h0, wdr_ref[0].astype(jnp.bfloat16))
        acc = acc + _bdot(h1, wdr_ref[1].astype(jnp.bfloat16))
        out_ref[...] += acc


@jax.jit
def _moe(hidden_states, W_gate, e_bias2, W_gate_up, W_down, Ws_gate_up,
         Ws_down3):
    grid = (T // TB, E // 2 + 1)
    cr = lambda t, p: jnp.clip(p - 1, 0, E // 2 - 1)
    return pl.pallas_call(
        _moe_body,
        grid=grid,
        in_specs=[
            pl.BlockSpec((TB, D), lambda t, p: (t, 0)),
            pl.BlockSpec((D, E), lambda t, p: (0, 0)),
            pl.BlockSpec((1, E), lambda t, p: (0, 0)),
            pl.BlockSpec((2, D, 2 * DFF),
                         lambda t, p: (cr(t, p), 0, 0)),
            pl.BlockSpec((D, 2 * DFF), lambda t, p: (0, 0)),
            pl.BlockSpec((D, 2 * DFF), lambda t, p: (0, 1)),
            pl.BlockSpec((2, DFF, D),
                         lambda t, p: (cr(t, p), 0, 0)),
            pl.BlockSpec((2, DFF, D), lambda t, p: (0, 0, 0)),
        ],
        out_specs=pl.BlockSpec((TB, D), lambda t, p: (t, 0)),
        out_shape=jax.ShapeDtypeStruct((T, D), jnp.float32),
        scratch_shapes=[pltpu.VMEM((TB, E), jnp.float32),
                        pltpu.VMEM((TB, D), jnp.bfloat16)],
        compiler_params=pltpu.CompilerParams(
            dimension_semantics=("arbitrary", "arbitrary"),
            vmem_limit_bytes=64 * 1024 * 1024),
    )(hidden_states, W_gate, e_bias2, W_gate_up,
      Ws_gate_up, Ws_gate_up, W_down, Ws_down3)


def kernel(hidden_states, W_gate, e_bias, W_gate_up, W_down, Ws_gate_up,
           Ws_down):
    e_bias2 = e_bias.reshape(1, E)
    Ws_down3 = Ws_down.reshape(NS, DFF, D)
    return _moe(hidden_states, W_gate, e_bias2, W_gate_up, W_down,
                Ws_gate_up, Ws_down3)


# speedup vs baseline: 6.2342x; 1.0009x over previous
import jax
import jax.numpy as jnp
from jax import lax
from jax.experimental import pallas as pl
from jax.experimental.pallas import tpu as pltpu

T = 2048
D = 1024
E = 8
DFF = 512
NG = 4
TG = 2
K = 2
NS = 2
RSF = 2.5

NEG = jnp.finfo(jnp.float32).min


def _first_k_mask(vals, k, triu):
    n = vals.shape[1]
    rem = vals
    sel = jnp.zeros_like(vals, dtype=jnp.bool_)
    for _ in range(k):
        m = jnp.max(rem, axis=1, keepdims=True)
        eq = rem == m
        cnt = lax.dot_general(
            eq.astype(jnp.float32), triu,
            (((1,), (0,)), ((), ())),
            precision=lax.Precision.HIGHEST,
        )
        first = jnp.logical_and(eq, cnt == 1.0)
        sel = jnp.logical_or(sel, first)
        rem = jnp.where(first, NEG, rem)
    return sel


def _routing(x, wg, eb):
    logits = lax.dot_general(
        x, wg, (((1,), (0,)), ((), ())), precision=lax.Precision.DEFAULT)
    scores = jax.nn.sigmoid(logits)
    sfc = scores + eb

    r8 = lax.broadcasted_iota(jnp.int32, (E, NG), 0)
    c8 = lax.broadcasted_iota(jnp.int32, (E, NG), 1)
    G = (r8 // (E // NG) == c8).astype(jnp.float32)
    gsum = lax.dot_general(
        sfc, G, (((1,), (0,)), ((), ())), precision=lax.Precision.HIGHEST)

    rg = lax.broadcasted_iota(jnp.int32, (NG, NG), 0)
    cg = lax.broadcasted_iota(jnp.int32, (NG, NG), 1)
    triu_g = (rg <= cg).astype(jnp.float32)
    gmask = _first_k_mask(gsum, TG, triu_g)

    smask = lax.dot_general(
        gmask.astype(jnp.float32), G.T, (((1,), (0,)), ((), ())),
        precision=lax.Precision.HIGHEST) > 0.5
    masked = jnp.where(smask, sfc, NEG)

    re_ = lax.broadcasted_iota(jnp.int32, (E, E), 0)
    ce_ = lax.broadcasted_iota(jnp.int32, (E, E), 1)
    triu_e = (re_ <= ce_).astype(jnp.float32)
    sel = _first_k_mask(masked, K, triu_e)

    w = jnp.where(sel, scores, 0.0)
    wsum = jnp.sum(w, axis=1, keepdims=True) + 1e-20
    return w / wsum * RSF


TB = 1024


def _bdot(a, b):
    return lax.dot_general(a, b, (((1,), (0,)), ((), ())),
                           preferred_element_type=jnp.float32)


def _moe_body(x_ref, wg_ref, eb_ref, wgur_ref, wsg_ref, wsu_ref,
              wdr_ref, wds_ref, out_ref, comb_ref, xb_ref):
    p = pl.program_id(1)

    @pl.when(p == 0)
    def _():
        xb0 = x_ref[...].astype(jnp.bfloat16)
        xb_ref[...] = xb0
        g0 = _bdot(xb0, wsg_ref[:, :DFF].astype(jnp.bfloat16))
        u0 = _bdot(xb0, wsu_ref[:, :DFF].astype(jnp.bfloat16))
        g1 = _bdot(xb0, wsg_ref[:, DFF:].astype(jnp.bfloat16))
        u1 = _bdot(xb0, wsu_ref[:, DFF:].astype(jnp.bfloat16))
        comb_ref[...] = _routing(x_ref[...], wg_ref[...], eb_ref[...])
        h0 = (g0 * jax.nn.sigmoid(g0) * u0).astype(jnp.bfloat16)
        h1 = (g1 * jax.nn.sigmoid(g1) * u1).astype(jnp.bfloat16)
        acc = _bdot(h0, wds_ref[0].astype(jnp.bfloat16))
        acc = acc + _bdot(h1, wds_ref[1].astype(jnp.bfloat16))
        out_ref[...] = acc

    xb = xb_ref[...]
    lane = lax.broadcasted_iota(jnp.int32, (TB, E), 1)

    @pl.when(p >= 1)
    def _():
        gu0 = _bdot(xb, wgur_ref[0].astype(jnp.bfloat16))
        gu1 = _bdot(xb, wgur_ref[1].astype(jnp.bfloat16))
        w0 = jnp.sum(jnp.where(lane == 2 * (p - 1), comb_ref[...], 0.0),
                     axis=1, keepdims=True)
        w1 = jnp.sum(jnp.where(lane == 2 * (p - 1) + 1, comb_ref[...], 0.0),
                     axis=1, keepdims=True)
        g0 = gu0[:, :DFF]
        u0 = gu0[:, DFF:]
        g1 = gu1[:, :DFF]
        u1 = gu1[:, DFF:]
        h0 = (g0 * jax.nn.sigmoid(g0) * u0 * w0).astype(jnp.bfloat16)
        h1 = (g1 * jax.nn.sigmoid(g1) * u1 * w1).astype(jnp.bfloat16)
        acc = _bdot(h0, wdr_ref[0].astype(jnp.bfloat16))
        acc = acc + _bdot(h1, wdr_ref[1].astype(jnp.bfloat16))
        out_ref[...] += acc


@jax.jit
def _moe(hidden_states, W_gate, e_bias2, W_gate_up, W_down, Ws_gate_up,
         Ws_down3):
    grid = (T // TB, E // 2 + 1)
    cr = lambda t, p: jnp.clip(p - 1, 0, E // 2 - 1)
    return pl.pallas_call(
        _moe_body,
        grid=grid,
        in_specs=[
            pl.BlockSpec((TB, D), lambda t, p: (t, 0)),
            pl.BlockSpec((D, E), lambda t, p: (0, 0)),
            pl.BlockSpec((1, E), lambda t, p: (0, 0)),
            pl.BlockSpec((2, D, 2 * DFF),
                         lambda t, p: (cr(t, p), 0, 0)),
            pl.BlockSpec((D, 2 * DFF), lambda t, p: (0, 0)),
            pl.BlockSpec((D, 2 * DFF), lambda t, p: (0, 1)),
            pl.BlockSpec((2, DFF, D),
                         lambda t, p: (cr(t, p), 0, 0)),
            pl.BlockSpec((2, DFF, D), lambda t, p: (0, 0, 0)),
        ],
        out_specs=pl.BlockSpec((TB, D), lambda t, p: (t, 0)),
        out_shape=jax.ShapeDtypeStruct((T, D), jnp.float32),
        scratch_shapes=[pltpu.VMEM((TB, E), jnp.float32),
                        pltpu.VMEM((TB, D), jnp.bfloat16)],
        compiler_params=pltpu.CompilerParams(
            dimension_semantics=("arbitrary", "arbitrary"),
            vmem_limit_bytes=64 * 1024 * 1024),
    )(hidden_states, W_gate, e_bias2, W_gate_up,
      Ws_gate_up, Ws_gate_up, W_down, Ws_down3)


def kernel(hidden_states, W_gate, e_bias, W_gate_up, W_down, Ws_gate_up,
           Ws_down):
    e_bias2 = e_bias.reshape(1, E)
    Ws_down3 = Ws_down.reshape(NS, DFF, D)
    return _moe(hidden_states, W_gate, e_bias2, W_gate_up, W_down,
                Ws_gate_up, Ws_down3)
